# Initial kernel scaffold; baseline (speedup 1.0000x reference)
import jax
import jax.numpy as jnp
from jax.experimental import pallas as pl


def kernel(given2rank1_stimulus_set, given8rank2_stimulus_set, percept_table):
    raise NotImplementedError("write your pallas kernel here")


# speedup vs baseline: 12.0771x; 12.0771x over previous
import functools

import jax
import jax.numpy as jnp
from jax import lax
from jax.experimental import pallas as pl
from jax.experimental.pallas import tpu as pltpu
from jax.experimental.pallas import tpu_sc as plsc

_N = 21
_L = 16
_TPAD = 72
_SPAD = 448


def _const(v):
    return jnp.full((_L,), v, jnp.int32)


def _build_sim_table(t_v, s_v):
    lane = lax.iota(jnp.int32, 16)

    def pair_group(g, carry):
        p = g * _L + lane
        i3 = (p // _N) * 3
        j3 = (p % _N) * 3
        d2 = jnp.zeros((_L,), jnp.float32)
        for c in range(3):
            diff = (plsc.load_gather(t_v, [i3 + _const(c)])
                    - plsc.load_gather(t_v, [j3 + _const(c)]))
            d2 = d2 + diff * diff
        y = plsc.bitcast(jnp.int32(0x5F3759DF) - (plsc.bitcast(d2, jnp.int32) >> 1),
                         jnp.float32)
        for _ in range(3):
            y = y * (1.5 - 0.5 * d2 * y * y)
        d = jnp.where(d2 > 0.0, d2 * y, 0.0)
        s_v[pl.ds(g * _L, _L)] = jnp.exp(-10.0 * d) + 0.001
        return carry

    lax.fori_loop(0, _SPAD // _L, pair_group, 0)


def _sc_rank_body(g2_hbm, g8_hbm, t_hbm, out1_hbm, out2_hbm,
                  t_v, s_v, g2_v, g8_v, o1_v, o2_v, bpw):
    wid = lax.axis_index("s") * 2 + lax.axis_index("c")
    base = wid * bpw
    pltpu.sync_copy(t_hbm, t_v)
    pltpu.sync_copy(g2_hbm.at[pl.ds(base * 3, bpw * 3)], g2_v)
    pltpu.sync_copy(g8_hbm.at[pl.ds(base * 9, bpw * 9)], g8_v)

    _build_sim_table(t_v, s_v)

    lane = lax.iota(jnp.int32, 16)

    def group(g, carry):
        rows = g * _L + lane
        r9 = rows * 9
        q21 = plsc.load_gather(g8_v, [r9]) * _N
        s = []
        for k in range(8):
            r = plsc.load_gather(g8_v, [r9 + _const(k + 1)])
            s.append(plsc.load_gather(s_v, [q21 + r]))
        total = s[0]
        for k in range(1, 8):
            total = total + s[k]
        rtot = 1.0 / total
        r56 = rows * 56
        col = 0
        for i in range(8):
            w_i = s[i] * rtot / (total - s[i])
            for j in range(8):
                if j == i:
                    continue
                plsc.store_scatter(o2_v, [r56 + _const(col)], w_i * s[j])
                col += 1
        r3 = rows * 3
        q21 = plsc.load_gather(g2_v, [r3]) * _N
        s1 = plsc.load_gather(s_v, [q21 + plsc.load_gather(g2_v, [r3 + _const(1)])])
        s2 = plsc.load_gather(s_v, [q21 + plsc.load_gather(g2_v, [r3 + _const(2)])])
        inv = 1.0 / (s1 + s2)
        r2 = rows * 2
        plsc.store_scatter(o1_v, [r2], s1 * inv)
        plsc.store_scatter(o1_v, [r2 + _const(1)], s2 * inv)
        return carry

    lax.fori_loop(0, bpw // _L, group, 0)

    pltpu.sync_copy(o1_v, out1_hbm.at[pl.ds(base * 2, bpw * 2)])
    pltpu.sync_copy(o2_v, out2_hbm.at[pl.ds(base * 56, bpw * 56)])


def kernel(given2rank1_stimulus_set, given8rank2_stimulus_set, percept_table):
    b = given2rank1_stimulus_set.shape[0]
    nw = 32
    bpw = b // nw
    assert b % (nw * _L) == 0

    t_flat = jnp.concatenate(
        [percept_table.reshape(_N * 3),
         jnp.zeros((_TPAD - _N * 3,), jnp.float32)])
    g2_flat = given2rank1_stimulus_set.reshape(b * 3)
    g8_flat = given8rank2_stimulus_set.reshape(b * 9)

    mesh = plsc.VectorSubcoreMesh(core_axis_name="c", subcore_axis_name="s")
    sc_call = pl.kernel(
        functools.partial(_sc_rank_body, bpw=bpw),
        out_type=(
            jax.ShapeDtypeStruct((b * 2,), jnp.float32),
            jax.ShapeDtypeStruct((b * 56,), jnp.float32),
        ),
        mesh=mesh,
        compiler_params=pltpu.CompilerParams(needs_layout_passes=False),
        scratch_types=[
            pltpu.VMEM((_TPAD,), jnp.float32),
            pltpu.VMEM((_SPAD,), jnp.float32),
            pltpu.VMEM((bpw * 3,), jnp.int32),
            pltpu.VMEM((bpw * 9,), jnp.int32),
            pltpu.VMEM((bpw * 2,), jnp.float32),
            pltpu.VMEM((bpw * 56,), jnp.float32),
        ],
    )
    out1, out2 = sc_call(g2_flat, g8_flat, t_flat)
    return (out1.reshape(b, 2), out2.reshape(b, 56))


# speedup vs baseline: 12.6928x; 1.0510x over previous
import functools

import jax
import jax.numpy as jnp
from jax import lax
from jax.experimental import pallas as pl
from jax.experimental.pallas import tpu as pltpu
from jax.experimental.pallas import tpu_sc as plsc

_N = 21
_L = 16
_TPAD = 72
_SPAD = 448
_CHUNKS = 4


def _const(v):
    return jnp.full((_L,), v, jnp.int32)


def _build_sim_table(t_v, s_v):
    lane = lax.iota(jnp.int32, 16)

    @functools.partial(plsc.parallel_loop, 0, _SPAD // _L)
    def pair_group(g):
        p = g * _L + lane
        i3 = (p // _N) * 3
        j3 = (p % _N) * 3
        d2 = jnp.zeros((_L,), jnp.float32)
        for c in range(3):
            diff = (plsc.load_gather(t_v, [i3 + _const(c)])
                    - plsc.load_gather(t_v, [j3 + _const(c)]))
            d2 = d2 + diff * diff
        y = plsc.bitcast(jnp.int32(0x5F3759DF) - (plsc.bitcast(d2, jnp.int32) >> 1),
                         jnp.float32)
        for _ in range(3):
            y = y * (1.5 - 0.5 * d2 * y * y)
        d = jnp.where(d2 > 0.0, d2 * y, 0.0)
        s_v[pl.ds(g * _L, _L)] = jnp.exp(-10.0 * d) + 0.001


def _sc_rank_body(g2_hbm, g8_hbm, t_hbm, out1_hbm, out2_hbm,
                  t_v, s_v, g2_v, g8_v, o1_v, o2_v, sem_in, sem_out, bpw):
    wid = lax.axis_index("s") * 2 + lax.axis_index("c")
    base = wid * bpw
    pltpu.sync_copy(t_hbm, t_v)
    cp2 = pltpu.async_copy(g2_hbm.at[pl.ds(base * 3, bpw * 3)], g2_v, sem_in)
    cp8 = pltpu.async_copy(g8_hbm.at[pl.ds(base * 9, bpw * 9)], g8_v, sem_in)

    _build_sim_table(t_v, s_v)

    cp2.wait()
    cp8.wait()

    lane = lax.iota(jnp.int32, 16)
    groups = bpw // _L
    gpc = groups // _CHUNKS
    out_cps = []

    for chunk in range(_CHUNKS):

        @functools.partial(plsc.parallel_loop, chunk * gpc, (chunk + 1) * gpc,
                           unroll=2)
        def group(g):
            rows = g * _L + lane
            r9 = rows * 9
            q21 = plsc.load_gather(g8_v, [r9]) * _N
            s = []
            for k in range(8):
                r = plsc.load_gather(g8_v, [r9 + _const(k + 1)])
                s.append(plsc.load_gather(s_v, [q21 + r]))
            total = s[0]
            for k in range(1, 8):
                total = total + s[k]
            rtot = 1.0 / total
            r56 = rows * 56
            col = 0
            for i in range(8):
                w_i = s[i] * rtot / (total - s[i])
                for j in range(8):
                    if j == i:
                        continue
                    plsc.store_scatter(o2_v, [r56 + _const(col)], w_i * s[j])
                    col += 1
            r3 = rows * 3
            q21 = plsc.load_gather(g2_v, [r3]) * _N
            s1 = plsc.load_gather(s_v, [q21 + plsc.load_gather(g2_v, [r3 + _const(1)])])
            s2 = plsc.load_gather(s_v, [q21 + plsc.load_gather(g2_v, [r3 + _const(2)])])
            inv = 1.0 / (s1 + s2)
            r2 = rows * 2
            plsc.store_scatter(o1_v, [r2], s1 * inv)
            plsc.store_scatter(o1_v, [r2 + _const(1)], s2 * inv)

        span = gpc * _L * 56
        out_cps.append(pltpu.async_copy(
            o2_v.at[pl.ds(chunk * span, span)],
            out2_hbm.at[pl.ds(base * 56 + chunk * span, span)],
            sem_out))

    pltpu.sync_copy(o1_v, out1_hbm.at[pl.ds(base * 2, bpw * 2)])
    for cp in out_cps:
        cp.wait()


def kernel(given2rank1_stimulus_set, given8rank2_stimulus_set, percept_table):
    b = given2rank1_stimulus_set.shape[0]
    nw = 32
    bpw = b // nw
    assert b % (nw * _L * _CHUNKS) == 0

    t_flat = jnp.concatenate(
        [percept_table.reshape(_N * 3),
         jnp.zeros((_TPAD - _N * 3,), jnp.float32)])
    g2_flat = given2rank1_stimulus_set.reshape(b * 3)
    g8_flat = given8rank2_stimulus_set.reshape(b * 9)

    mesh = plsc.VectorSubcoreMesh(core_axis_name="c", subcore_axis_name="s")
    sc_call = pl.kernel(
        functools.partial(_sc_rank_body, bpw=bpw),
        out_type=(
            jax.ShapeDtypeStruct((b * 2,), jnp.float32),
            jax.ShapeDtypeStruct((b * 56,), jnp.float32),
        ),
        mesh=mesh,
        compiler_params=pltpu.CompilerParams(needs_layout_passes=False),
        scratch_types=[
            pltpu.VMEM((_TPAD,), jnp.float32),
            pltpu.VMEM((_SPAD,), jnp.float32),
            pltpu.VMEM((bpw * 3,), jnp.int32),
            pltpu.VMEM((bpw * 9,), jnp.int32),
            pltpu.VMEM((bpw * 2,), jnp.float32),
            pltpu.VMEM((bpw * 56,), jnp.float32),
            pltpu.SemaphoreType.DMA,
            pltpu.SemaphoreType.DMA,
        ],
    )
    out1, out2 = sc_call(g2_flat, g8_flat, t_flat)
    return (out1.reshape(b, 2), out2.reshape(b, 56))
